# baseline (speedup 1.0000x reference)
import jax
import jax.numpy as jnp
from jax.experimental import pallas as pl
from jax.experimental.pallas import tpu as pltpu
from jax.experimental.pallas import tpu_sc as plsc

_LANES = 16
_WINDOW = 128


def kernel(lyrics_ids, mask, table):
    B, S = lyrics_ids.shape
    V, D = table.shape
    N = B * S

    ids = lyrics_ids.reshape(1, N).astype(jnp.int32)
    msk = mask.reshape(1, N).astype(jnp.int32)

    mesh = plsc.VectorSubcoreMesh(
        core_axis_name="core", subcore_axis_name="subcore"
    )

    @pl.kernel(
        out_type=jax.ShapeDtypeStruct((N, D), table.dtype),
        mesh=mesh,
        scratch_types=[pltpu.VMEM((1, _WINDOW), jnp.int32)],
    )
    def sc_gather(table_hbm, ids_hbm, mask_hbm, o_hbm, idx_scratch):
        def body(i_vmem, m_vmem, o_vmem):
            @pl.loop(0, _WINDOW, step=_LANES)
            def _(c):
                slc = (pl.ds(0, 1), pl.ds(c, _LANES))
                idx_scratch.at[*slc][...] = (
                    i_vmem.at[*slc][...] * m_vmem.at[*slc][...]
                )

            pltpu.sync_copy(table_hbm.at[idx_scratch.at[0]], o_vmem)

        pltpu.emit_pipeline(
            body,
            grid=(N // _WINDOW,),
            in_specs=[
                pl.BlockSpec((1, _WINDOW), lambda i: (0, i)),
                pl.BlockSpec((1, _WINDOW), lambda i: (0, i)),
            ],
            out_specs=[pl.BlockSpec((_WINDOW, D), lambda i: (i, 0))],
            core_axis_name=("core", "subcore"),
            dimension_semantics=(pltpu.PARALLEL,),
        )(ids_hbm, mask_hbm, o_hbm)

    out = sc_gather(table, ids, msk)
    return out.reshape(B, S, D)
